# baseline (speedup 1.0000x reference)
import functools

import jax
import jax.numpy as jnp
from jax.experimental import pallas as pl

_SPLITS = 4


def _tree_lstm_block(x_ref, *refs, k: int, h_dim: int, half_n: int):
    mh_refs = refs[:_SPLITS]
    mc_refs = refs[_SPLITS:2 * _SPLITS]
    (Wfh_t_ref, bfh_ref, Ufh_t_ref,
     Wiou_t_ref, biou_ref, Uiou_t_ref,
     h_ref, c_ref) = refs[2 * _SPLITS:]

    Wfh_t = Wfh_t_ref[...]
    bfh = bfh_ref[...]
    Ufh_t = Ufh_t_ref[...]
    Wiou_t = Wiou_t_ref[...]
    biou = biou_ref[...]
    Uiou_t = Uiou_t_ref[...]

    for s in range(_SPLITS):
        xb = x_ref[pl.ds(s * half_n, half_n), :]
        mh = mh_refs[s][...]
        mc = mc_refs[s][...]

        h_tild = jnp.sum(mh, axis=1)
        mc_sum = jnp.sum(mc, axis=1)

        wx_h = jnp.dot(xb, Wfh_t, preferred_element_type=jnp.float32) + bfh
        uh_h = jnp.dot(mh.reshape(half_n * k, h_dim), Ufh_t,
                       preferred_element_type=jnp.float32)
        t = jnp.tanh(uh_h.reshape(half_n, k, h_dim) + wx_h[:, None, :])
        c_tild = 0.5 * (jnp.sum(t * mc, axis=1) + mc_sum)

        iou = (jnp.dot(xb, Wiou_t, preferred_element_type=jnp.float32)
               + jnp.dot(h_tild, Uiou_t, preferred_element_type=jnp.float32)
               + biou)
        i_g = 0.5 * jnp.tanh(iou[:, :h_dim]) + 0.5
        o_g = 0.5 * jnp.tanh(iou[:, h_dim:2 * h_dim]) + 0.5
        u_g = jnp.tanh(iou[:, 2 * h_dim:])

        c = i_g * u_g + c_tild
        h_ref[pl.ds(s * half_n, half_n), :] = o_g * jnp.tanh(c)
        c_ref[pl.ds(s * half_n, half_n), :] = c


def kernel(x, msgs_h, msgs_c, W_iou, b_iou, U_iou, b_uiou, W_f, b_wf, U_f, b_uf):
    n, k, h_dim = msgs_h.shape
    x_dim = x.shape[1]

    block_n = 400
    half_n = block_n // _SPLITS
    assert n % block_n == 0
    grid = (n // block_n,)

    iou_scale = jnp.concatenate(
        [jnp.full((2 * h_dim,), 0.5, jnp.float32),
         jnp.ones((h_dim,), jnp.float32)])

    full = lambda i: (0, 0)

    def msg_spec(s):
        return pl.BlockSpec((half_n, k, h_dim),
                            lambda i, s=s: (_SPLITS * i + s, 0, 0))

    body = functools.partial(_tree_lstm_block, k=k, h_dim=h_dim,
                             half_n=half_n)

    h, c = pl.pallas_call(
        body,
        grid=grid,
        in_specs=(
            [pl.BlockSpec((block_n, x_dim), lambda i: (i, 0))]
            + [msg_spec(s) for s in range(_SPLITS)]
            + [msg_spec(s) for s in range(_SPLITS)]
            + [
                pl.BlockSpec((x_dim, h_dim), full),
                pl.BlockSpec((1, h_dim), full),
                pl.BlockSpec((h_dim, h_dim), full),
                pl.BlockSpec((x_dim, 3 * h_dim), full),
                pl.BlockSpec((1, 3 * h_dim), full),
                pl.BlockSpec((h_dim, 3 * h_dim), full),
            ]
        ),
        out_specs=[
            pl.BlockSpec((block_n, h_dim), lambda i: (i, 0)),
            pl.BlockSpec((block_n, h_dim), lambda i: (i, 0)),
        ],
        out_shape=[
            jax.ShapeDtypeStruct((n, h_dim), jnp.float32),
            jax.ShapeDtypeStruct((n, h_dim), jnp.float32),
        ],
    )(
        x,
        *([msgs_h] * _SPLITS),
        *([msgs_c] * _SPLITS),
        0.5 * W_f.T, (0.5 * (b_wf + b_uf)).reshape(1, h_dim),
        0.5 * U_f.T,
        W_iou.T * iou_scale, ((b_iou + b_uiou) * iou_scale).reshape(1, 3 * h_dim),
        U_iou.T * iou_scale,
    )
    return (h, c)


# speedup vs baseline: 1.0038x; 1.0038x over previous
import functools

import jax
import jax.numpy as jnp
from jax.experimental import pallas as pl
from jax.experimental.pallas import tpu as pltpu

_SPLITS = 2


def _tree_lstm_block(x_ref, *refs, k: int, h_dim: int, half_n: int):
    mh_refs = refs[:_SPLITS]
    mc_refs = refs[_SPLITS:2 * _SPLITS]
    (Wfh_t_ref, bfh_ref, Ufh_t_ref,
     Wiou_t_ref, biou_ref, Uiou_t_ref,
     h_ref, c_ref) = refs[2 * _SPLITS:]

    Wfh_t = Wfh_t_ref[...]
    bfh = bfh_ref[...]
    Ufh_t = Ufh_t_ref[...]
    Wiou_t = Wiou_t_ref[...]
    biou = biou_ref[...]
    Uiou_t = Uiou_t_ref[...]

    for s in range(_SPLITS):
        xb = x_ref[pl.ds(s * half_n, half_n), :]
        mh = mh_refs[s][...]
        mc = mc_refs[s][...]

        h_tild = jnp.sum(mh, axis=1)
        mc_sum = jnp.sum(mc, axis=1)

        wx_h = jnp.dot(xb, Wfh_t, preferred_element_type=jnp.float32) + bfh
        uh_h = jnp.dot(mh.reshape(half_n * k, h_dim), Ufh_t,
                       preferred_element_type=jnp.float32)
        t = jnp.tanh(uh_h.reshape(half_n, k, h_dim) + wx_h[:, None, :])
        c_tild = 0.5 * (jnp.sum(t * mc, axis=1) + mc_sum)

        iou = (jnp.dot(xb, Wiou_t, preferred_element_type=jnp.float32)
               + jnp.dot(h_tild, Uiou_t, preferred_element_type=jnp.float32)
               + biou)
        i_g = 0.5 * jnp.tanh(iou[:, :h_dim]) + 0.5
        o_g = 0.5 * jnp.tanh(iou[:, h_dim:2 * h_dim]) + 0.5
        u_g = jnp.tanh(iou[:, 2 * h_dim:])

        c = i_g * u_g + c_tild
        h_ref[pl.ds(s * half_n, half_n), :] = o_g * jnp.tanh(c)
        c_ref[pl.ds(s * half_n, half_n), :] = c


def kernel(x, msgs_h, msgs_c, W_iou, b_iou, U_iou, b_uiou, W_f, b_wf, U_f, b_uf):
    n, k, h_dim = msgs_h.shape
    x_dim = x.shape[1]

    block_n = 400
    half_n = block_n // _SPLITS
    assert n % block_n == 0
    grid = (n // block_n,)

    iou_scale = jnp.concatenate(
        [jnp.full((2 * h_dim,), 0.5, jnp.float32),
         jnp.ones((h_dim,), jnp.float32)])

    full = lambda i: (0, 0)

    def msg_spec(s):
        return pl.BlockSpec((half_n, k, h_dim),
                            lambda i, s=s: (_SPLITS * i + s, 0, 0))

    body = functools.partial(_tree_lstm_block, k=k, h_dim=h_dim,
                             half_n=half_n)

    h, c = pl.pallas_call(
        body,
        grid=grid,
        compiler_params=pltpu.CompilerParams(
            dimension_semantics=("parallel",)),
        in_specs=(
            [pl.BlockSpec((block_n, x_dim), lambda i: (i, 0))]
            + [msg_spec(s) for s in range(_SPLITS)]
            + [msg_spec(s) for s in range(_SPLITS)]
            + [
                pl.BlockSpec((x_dim, h_dim), full),
                pl.BlockSpec((1, h_dim), full),
                pl.BlockSpec((h_dim, h_dim), full),
                pl.BlockSpec((x_dim, 3 * h_dim), full),
                pl.BlockSpec((1, 3 * h_dim), full),
                pl.BlockSpec((h_dim, 3 * h_dim), full),
            ]
        ),
        out_specs=[
            pl.BlockSpec((block_n, h_dim), lambda i: (i, 0)),
            pl.BlockSpec((block_n, h_dim), lambda i: (i, 0)),
        ],
        out_shape=[
            jax.ShapeDtypeStruct((n, h_dim), jnp.float32),
            jax.ShapeDtypeStruct((n, h_dim), jnp.float32),
        ],
    )(
        x,
        *([msgs_h] * _SPLITS),
        *([msgs_c] * _SPLITS),
        0.5 * W_f.T, (0.5 * (b_wf + b_uf)).reshape(1, h_dim),
        0.5 * U_f.T,
        W_iou.T * iou_scale, ((b_iou + b_uiou) * iou_scale).reshape(1, 3 * h_dim),
        U_iou.T * iou_scale,
    )
    return (h, c)
